# baseline (speedup 1.0000x reference)
import jax
import jax.numpy as jnp
from jax import lax
from jax.experimental import pallas as pl
from jax.experimental.pallas import tpu as pltpu
from jax.experimental.pallas import tpu_sc as plsc

N = 10000
E = 320000
DIN = 128
H = 256
HALF = 128
OUT = 128
L = 3

NC = 2
NS = 16
K = 64
NCHUNK = E // K
NCH_A, NCH_B = 320, 200
NCH0_A, NCH0_B = 160, 40
RND = 64
ROWS_PER_TILE = 632
N_PAD = NS * ROWS_PER_TILE


def _zero_slab(pooled_sp, zbuf, s):
    def zero_body(i, carry):
        for cc in range(8):
            zbuf[i, pl.ds(cc * 16, 16)] = jnp.zeros((16,), jnp.float32)
        return carry
    lax.fori_loop(0, K, zero_body, 0)
    done = 0
    while done < ROWS_PER_TILE:
        m = min(K, ROWS_PER_TILE - done)
        pltpu.sync_copy(zbuf.at[pl.ds(0, m)],
                        pooled_sp.at[pl.ds(s * ROWS_PER_TILE + done, m)])
        done += m


def _round_segsum(h_hbm, pooled_sp, src_all, dst_all, rows, sem_g, sem_s,
                  nch):
    B = 4

    def gstart(b, c):
        pltpu.async_copy(h_hbm.at[src_all.at[c]], rows[b], sem_g[b])

    def gwait(b, c):
        pltpu.make_async_copy(h_hbm.at[src_all.at[c]], rows[b], sem_g[b]).wait()

    def sstart(b, c):
        pltpu.async_copy(rows[b], pooled_sp.at[pl.ds(b * K, K)], sem_s[b])

    def swait(b, c):
        pltpu.make_async_copy(rows[b], pooled_sp.at[pl.ds(b * K, K)],
                              sem_s[b]).wait()

    gstart(0, 0)
    gstart(1, 1)
    gwait(0, 0)
    sstart(0, 0)
    gstart(2, 2)
    gwait(1, 1)
    sstart(1, 1)
    gstart(3, 3)

    def body(i, carry):
        for j in range(B):
            c = 2 + B * i + j
            b = (2 + j) % B
            gwait(b, c)
            sstart(b, c)
            swait((b + 2) % B, c - 2)
            gstart((b + 2) % B, c + 2)
        return carry

    lax.fori_loop(0, (nch - 4) // B, body, 0)
    c = nch - 2
    b = c % B
    gwait(b, c)
    sstart(b, c)
    swait((b + 2) % B, c - 2)
    c = nch - 1
    b = c % B
    gwait(b, c)
    sstart(b, c)
    swait((b + 2) % B, c - 2)
    swait((nch - 2) % B, nch - 2)
    swait((nch - 1) % B, nch - 1)


def _pipelined_segsum(h_hbm, src2d, dst2d, pooled_sp, src_all, dst_all,
                      rows, sem_g, sem_s, base, nch):
    done = 0
    while done < nch:
        rn = min(RND, nch - done)
        pltpu.sync_copy(src2d.at[pl.ds(base + done, rn)],
                        src_all.at[pl.ds(0, rn)])
        pltpu.sync_copy(dst2d.at[pl.ds(base + done, rn)],
                        dst_all.at[pl.ds(0, rn)])
        _round_segsum(h_hbm, pooled_sp, src_all, dst_all, rows, sem_g, sem_s,
                      rn)
        done += rn


def _segsum_body(h0_hbm, h1_hbm, src2d, dst2d, out0_hbm, out1_hbm,
                 pooled_sp, src_all, dst_all, rows0, rows1, rows2, rows3,
                 sg0, sg1, sg2, sg3, ss0, ss1, ss2, ss3):
    c = lax.axis_index("c")
    s = lax.axis_index("s")
    rows = (rows0, rows1, rows2, rows3)
    sem_g = (sg0, sg1, sg2, sg3)
    sem_s = (ss0, ss1, ss2, ss3)

    plsc.subcore_barrier()

    def run(h_hbm):
        @pl.when(s < NS - 1)
        def _():
            _pipelined_segsum(h_hbm, src2d, dst2d, pooled_sp, src_all,
                              dst_all, rows, sem_g, sem_s, s * NCH_A, NCH_A)

        @pl.when(s == NS - 1)
        def _():
            _pipelined_segsum(h_hbm, src2d, dst2d, pooled_sp, src_all,
                              dst_all, rows, sem_g, sem_s, (NS - 1) * NCH_A,
                              NCH_B)

    run(h0_hbm)

    plsc.subcore_barrier()


_segsum = pl.kernel(
    _segsum_body,
    out_type=(
        jax.ShapeDtypeStruct((N_PAD, HALF), jnp.float32),
        jax.ShapeDtypeStruct((N_PAD, HALF), jnp.float32),
    ),
    mesh=plsc.VectorSubcoreMesh(core_axis_name="c", subcore_axis_name="s"),
    scratch_types=[
        pltpu.VMEM_SHARED((2048, H), jnp.float32),
        pltpu.VMEM((RND, K), jnp.int32),
        pltpu.VMEM((RND, K), jnp.int32),
        pltpu.VMEM((K, H), jnp.float32),
        pltpu.VMEM((K, H), jnp.float32),
        pltpu.VMEM((K, H), jnp.float32),
        pltpu.VMEM((K, H), jnp.float32),
        pltpu.SemaphoreType.DMA,
        pltpu.SemaphoreType.DMA,
        pltpu.SemaphoreType.DMA,
        pltpu.SemaphoreType.DMA,
        pltpu.SemaphoreType.DMA,
        pltpu.SemaphoreType.DMA,
        pltpu.SemaphoreType.DMA,
        pltpu.SemaphoreType.DMA,
    ],
)


def _segsum0_body(h0_hbm, src2d, dst2d, outa_hbm, outb_hbm,
                  pooled_sp, src_all, dst_all, rows0, rows1, rows2, rows3,
                  sg0, sg1, sg2, sg3, ss0, ss1, ss2, ss3):
    c = lax.axis_index("c")
    s = lax.axis_index("s")
    w = c * NS + s
    rows = (rows0, rows1, rows2, rows3)
    sem_g = (sg0, sg1, sg2, sg3)
    sem_s = (ss0, ss1, ss2, ss3)

    plsc.subcore_barrier()

    @pl.when(w < NC * NS - 1)
    def _():
        _pipelined_segsum(h0_hbm, src2d, dst2d, pooled_sp, src_all, dst_all,
                          rows, sem_g, sem_s, w * NCH0_A, NCH0_A)

    @pl.when(w == NC * NS - 1)
    def _():
        _pipelined_segsum(h0_hbm, src2d, dst2d, pooled_sp, src_all, dst_all,
                          rows, sem_g, sem_s, (NC * NS - 1) * NCH0_A, NCH0_B)

    plsc.subcore_barrier()


_segsum0 = pl.kernel(
    _segsum0_body,
    out_type=(
        jax.ShapeDtypeStruct((N_PAD, HALF), jnp.float32),
        jax.ShapeDtypeStruct((N_PAD, HALF), jnp.float32),
    ),
    mesh=plsc.VectorSubcoreMesh(core_axis_name="c", subcore_axis_name="s"),
    scratch_types=[
        pltpu.VMEM_SHARED((2048, H), jnp.float32),
        pltpu.VMEM((RND, K), jnp.int32),
        pltpu.VMEM((RND, K), jnp.int32),
        pltpu.VMEM((K, H), jnp.float32),
        pltpu.VMEM((K, H), jnp.float32),
        pltpu.VMEM((K, H), jnp.float32),
        pltpu.VMEM((K, H), jnp.float32),
        pltpu.SemaphoreType.DMA,
        pltpu.SemaphoreType.DMA,
        pltpu.SemaphoreType.DMA,
        pltpu.SemaphoreType.DMA,
        pltpu.SemaphoreType.DMA,
        pltpu.SemaphoreType.DMA,
        pltpu.SemaphoreType.DMA,
        pltpu.SemaphoreType.DMA,
    ],
)


def _bn_relu(comb, gamma, beta):
    mean = jnp.mean(comb, axis=0, keepdims=True)
    d = comb - mean
    var = jnp.mean(d * d, axis=0, keepdims=True)
    hn = d * lax.rsqrt(var + 1e-5) * gamma + beta
    return jnp.maximum(hn, 0.0)


def _combine0_body(x_ref, pa_ref, pb_ref, vw_ref, aw_ref, bias_ref,
                   gamma_ref, beta_ref, o0_ref, o1_ref):
    p = pa_ref[:N, :] + pb_ref[:N, :]
    comb = (jnp.dot(x_ref[...], vw_ref[...], preferred_element_type=jnp.float32)
            + jnp.dot(p, aw_ref[...], preferred_element_type=jnp.float32)
            + bias_ref[...])
    h_new = _bn_relu(comb, gamma_ref[...], beta_ref[...])
    o0_ref[...] = h_new[:, :HALF]
    o1_ref[...] = h_new[:, HALF:]


_combine0 = pl.pallas_call(
    _combine0_body,
    out_shape=(
        jax.ShapeDtypeStruct((N, HALF), jnp.float32),
        jax.ShapeDtypeStruct((N, HALF), jnp.float32),
    ),
)


def _combine_body(h0_ref, h1_ref, p0_ref, p1_ref, vw_ref, aw_ref, bias_ref,
                  gamma_ref, beta_ref, o0_ref, o1_ref):
    h = jnp.concatenate([h0_ref[...], h1_ref[...]], axis=1)
    p = jnp.concatenate([p0_ref[:N, :], p1_ref[:N, :]], axis=1)
    comb = (jnp.dot(h, vw_ref[...], preferred_element_type=jnp.float32)
            + jnp.dot(p, aw_ref[...], preferred_element_type=jnp.float32)
            + bias_ref[...])
    h_new = _bn_relu(comb, gamma_ref[...], beta_ref[...])
    o0_ref[...] = h_new[:, :HALF]
    o1_ref[...] = h_new[:, HALF:]


_combine = pl.pallas_call(
    _combine_body,
    out_shape=(
        jax.ShapeDtypeStruct((N, HALF), jnp.float32),
        jax.ShapeDtypeStruct((N, HALF), jnp.float32),
    ),
)


def _head_body(h0_ref, h1_ref, w_ref, b_ref, o_ref):
    h = jnp.concatenate([h0_ref[...], h1_ref[...]], axis=1)
    o_ref[...] = (jnp.dot(h, w_ref[...], preferred_element_type=jnp.float32)
                  + b_ref[...])


_head = pl.pallas_call(
    _head_body,
    out_shape=jax.ShapeDtypeStruct((N, OUT), jnp.float32),
)


def kernel(x, edge_index, V_w, V_b, A_w, A_b, bn_gamma, bn_beta, pred_w, pred_b):
    dst2d = edge_index[0].reshape(NCHUNK, K)
    src2d = edge_index[1].reshape(NCHUNK, K)

    pa, pb = _segsum0(jnp.concatenate([x, x], axis=1), src2d, dst2d)
    bias0 = (V_b[0] + A_b[0]).reshape(1, H)
    h0, h1 = _combine0(x, pa, pb, V_w[0, :HALF, :], A_w[0, :HALF, :], bias0,
                       bn_gamma[0].reshape(1, H), bn_beta[0].reshape(1, H))
    for l in range(1, L):
        p0, p1 = _segsum(jnp.concatenate([h0, h1], axis=1), h1, src2d, dst2d)
        bias = (V_b[l] + A_b[l]).reshape(1, H)
        h0, h1 = _combine(h0, h1, p0, p1, V_w[l], A_w[l], bias,
                          bn_gamma[l].reshape(1, H), bn_beta[l].reshape(1, H))
    return _head(h0, h1, pred_w, pred_b.reshape(1, OUT))


# speedup vs baseline: 1.4667x; 1.4667x over previous
import jax
import jax.numpy as jnp
from jax import lax
from jax.experimental import pallas as pl
from jax.experimental.pallas import tpu as pltpu
from jax.experimental.pallas import tpu_sc as plsc

N = 10000
E = 320000
DIN = 128
H = 256
HALF = 128
OUT = 128
L = 3

NC = 2
NS = 16
K = 64
NCHUNK = E // K
NCH_A, NCH_B = 320, 200
NCH0_A, NCH0_B = 160, 40
RND = 64
ROWS_PER_TILE = 632
N_PAD = NS * ROWS_PER_TILE


def _zero_slab(pooled_sp, zbuf, s):
    def zero_body(i, carry):
        for cc in range(8):
            zbuf[i, pl.ds(cc * 16, 16)] = jnp.zeros((16,), jnp.float32)
        return carry
    lax.fori_loop(0, K, zero_body, 0)
    done = 0
    while done < ROWS_PER_TILE:
        m = min(K, ROWS_PER_TILE - done)
        pltpu.sync_copy(zbuf.at[pl.ds(0, m)],
                        pooled_sp.at[pl.ds(s * ROWS_PER_TILE + done, m)])
        done += m


def _round_segsum(h_hbm, pooled_sp, src_all, dst_all, rows, sem_g, sem_s,
                  nch):
    B = 4

    def gstart(b, c):
        pltpu.async_copy(h_hbm.at[src_all.at[c]], rows[b], sem_g[b])

    def gwait(b, c):
        pltpu.make_async_copy(h_hbm.at[src_all.at[c]], rows[b], sem_g[b]).wait()

    def sstart(b, c):
        pltpu.async_copy(rows[b], pooled_sp.at[dst_all.at[c]], sem_s[b],
                         add=True)

    def swait(b, c):
        pltpu.make_async_copy(rows[b], pooled_sp.at[dst_all.at[c]],
                              sem_s[b]).wait()

    gstart(0, 0)
    gstart(1, 1)
    gwait(0, 0)
    sstart(0, 0)
    gstart(2, 2)
    gwait(1, 1)
    sstart(1, 1)
    gstart(3, 3)

    def body(i, carry):
        for j in range(B):
            c = 2 + B * i + j
            b = (2 + j) % B
            gwait(b, c)
            sstart(b, c)
            swait((b + 2) % B, c - 2)
            gstart((b + 2) % B, c + 2)
        return carry

    lax.fori_loop(0, (nch - 4) // B, body, 0)
    c = nch - 2
    b = c % B
    gwait(b, c)
    sstart(b, c)
    swait((b + 2) % B, c - 2)
    c = nch - 1
    b = c % B
    gwait(b, c)
    sstart(b, c)
    swait((b + 2) % B, c - 2)
    swait((nch - 2) % B, nch - 2)
    swait((nch - 1) % B, nch - 1)


def _pipelined_segsum(h_hbm, src2d, dst2d, pooled_sp, src_all, dst_all,
                      rows, sem_g, sem_s, base, nch):
    done = 0
    while done < nch:
        rn = min(RND, nch - done)
        pltpu.sync_copy(src2d.at[pl.ds(base + done, rn)],
                        src_all.at[pl.ds(0, rn)])
        pltpu.sync_copy(dst2d.at[pl.ds(base + done, rn)],
                        dst_all.at[pl.ds(0, rn)])
        _round_segsum(h_hbm, pooled_sp, src_all, dst_all, rows, sem_g, sem_s,
                      rn)
        done += rn


def _segsum_body(h0_hbm, h1_hbm, src2d, dst2d, out0_hbm, out1_hbm,
                 pooled_sp, src_all, dst_all, rows0, rows1, rows2, rows3,
                 sg0, sg1, sg2, sg3, ss0, ss1, ss2, ss3):
    c = lax.axis_index("c")
    s = lax.axis_index("s")
    rows = (rows0, rows1, rows2, rows3)
    sem_g = (sg0, sg1, sg2, sg3)
    sem_s = (ss0, ss1, ss2, ss3)

    _zero_slab(pooled_sp, rows0, s)
    plsc.subcore_barrier()

    def run(h_hbm):
        @pl.when(s < NS - 1)
        def _():
            _pipelined_segsum(h_hbm, src2d, dst2d, pooled_sp, src_all,
                              dst_all, rows, sem_g, sem_s, s * NCH_A, NCH_A)

        @pl.when(s == NS - 1)
        def _():
            _pipelined_segsum(h_hbm, src2d, dst2d, pooled_sp, src_all,
                              dst_all, rows, sem_g, sem_s, (NS - 1) * NCH_A,
                              NCH_B)

    @pl.when(c == 0)
    def _():
        run(h0_hbm)

    @pl.when(c == 1)
    def _():
        run(h1_hbm)

    plsc.subcore_barrier()

    @pl.when(c == 0)
    def _():
        pltpu.sync_copy(pooled_sp.at[pl.ds(s * ROWS_PER_TILE, ROWS_PER_TILE)],
                        out0_hbm.at[pl.ds(s * ROWS_PER_TILE, ROWS_PER_TILE)])

    @pl.when(c == 1)
    def _():
        pltpu.sync_copy(pooled_sp.at[pl.ds(s * ROWS_PER_TILE, ROWS_PER_TILE)],
                        out1_hbm.at[pl.ds(s * ROWS_PER_TILE, ROWS_PER_TILE)])


_segsum = pl.kernel(
    _segsum_body,
    out_type=(
        jax.ShapeDtypeStruct((N_PAD, HALF), jnp.float32),
        jax.ShapeDtypeStruct((N_PAD, HALF), jnp.float32),
    ),
    mesh=plsc.VectorSubcoreMesh(core_axis_name="c", subcore_axis_name="s"),
    scratch_types=[
        pltpu.VMEM_SHARED((N_PAD, HALF), jnp.float32),
        pltpu.VMEM((RND, K), jnp.int32),
        pltpu.VMEM((RND, K), jnp.int32),
        pltpu.VMEM((K, HALF), jnp.float32),
        pltpu.VMEM((K, HALF), jnp.float32),
        pltpu.VMEM((K, HALF), jnp.float32),
        pltpu.VMEM((K, HALF), jnp.float32),
        pltpu.SemaphoreType.DMA,
        pltpu.SemaphoreType.DMA,
        pltpu.SemaphoreType.DMA,
        pltpu.SemaphoreType.DMA,
        pltpu.SemaphoreType.DMA,
        pltpu.SemaphoreType.DMA,
        pltpu.SemaphoreType.DMA,
        pltpu.SemaphoreType.DMA,
    ],
)


def _segsum0_body(h0_hbm, src2d, dst2d, outa_hbm, outb_hbm,
                  pooled_sp, src_all, dst_all, rows0, rows1, rows2, rows3,
                  sg0, sg1, sg2, sg3, ss0, ss1, ss2, ss3):
    c = lax.axis_index("c")
    s = lax.axis_index("s")
    w = c * NS + s
    rows = (rows0, rows1, rows2, rows3)
    sem_g = (sg0, sg1, sg2, sg3)
    sem_s = (ss0, ss1, ss2, ss3)

    _zero_slab(pooled_sp, rows0, s)
    plsc.subcore_barrier()

    @pl.when(w < NC * NS - 1)
    def _():
        _pipelined_segsum(h0_hbm, src2d, dst2d, pooled_sp, src_all, dst_all,
                          rows, sem_g, sem_s, w * NCH0_A, NCH0_A)

    @pl.when(w == NC * NS - 1)
    def _():
        _pipelined_segsum(h0_hbm, src2d, dst2d, pooled_sp, src_all, dst_all,
                          rows, sem_g, sem_s, (NC * NS - 1) * NCH0_A, NCH0_B)

    plsc.subcore_barrier()

    @pl.when(c == 0)
    def _():
        pltpu.sync_copy(pooled_sp.at[pl.ds(s * ROWS_PER_TILE, ROWS_PER_TILE)],
                        outa_hbm.at[pl.ds(s * ROWS_PER_TILE, ROWS_PER_TILE)])

    @pl.when(c == 1)
    def _():
        pltpu.sync_copy(pooled_sp.at[pl.ds(s * ROWS_PER_TILE, ROWS_PER_TILE)],
                        outb_hbm.at[pl.ds(s * ROWS_PER_TILE, ROWS_PER_TILE)])


_segsum0 = pl.kernel(
    _segsum0_body,
    out_type=(
        jax.ShapeDtypeStruct((N_PAD, HALF), jnp.float32),
        jax.ShapeDtypeStruct((N_PAD, HALF), jnp.float32),
    ),
    mesh=plsc.VectorSubcoreMesh(core_axis_name="c", subcore_axis_name="s"),
    scratch_types=[
        pltpu.VMEM_SHARED((N_PAD, HALF), jnp.float32),
        pltpu.VMEM((RND, K), jnp.int32),
        pltpu.VMEM((RND, K), jnp.int32),
        pltpu.VMEM((K, HALF), jnp.float32),
        pltpu.VMEM((K, HALF), jnp.float32),
        pltpu.VMEM((K, HALF), jnp.float32),
        pltpu.VMEM((K, HALF), jnp.float32),
        pltpu.SemaphoreType.DMA,
        pltpu.SemaphoreType.DMA,
        pltpu.SemaphoreType.DMA,
        pltpu.SemaphoreType.DMA,
        pltpu.SemaphoreType.DMA,
        pltpu.SemaphoreType.DMA,
        pltpu.SemaphoreType.DMA,
        pltpu.SemaphoreType.DMA,
    ],
)


def _bn_relu(comb, gamma, beta):
    mean = jnp.mean(comb, axis=0, keepdims=True)
    d = comb - mean
    var = jnp.mean(d * d, axis=0, keepdims=True)
    hn = d * lax.rsqrt(var + 1e-5) * gamma + beta
    return jnp.maximum(hn, 0.0)


def _combine0_body(x_ref, pa_ref, pb_ref, vw_ref, aw_ref, bias_ref,
                   gamma_ref, beta_ref, o0_ref, o1_ref):
    p = pa_ref[:N, :] + pb_ref[:N, :]
    comb = (jnp.dot(x_ref[...], vw_ref[...], preferred_element_type=jnp.float32)
            + jnp.dot(p, aw_ref[...], preferred_element_type=jnp.float32)
            + bias_ref[...])
    h_new = _bn_relu(comb, gamma_ref[...], beta_ref[...])
    o0_ref[...] = h_new[:, :HALF]
    o1_ref[...] = h_new[:, HALF:]


_combine0 = pl.pallas_call(
    _combine0_body,
    out_shape=(
        jax.ShapeDtypeStruct((N, HALF), jnp.float32),
        jax.ShapeDtypeStruct((N, HALF), jnp.float32),
    ),
)


def _combine_body(h0_ref, h1_ref, p0_ref, p1_ref, vw_ref, aw_ref, bias_ref,
                  gamma_ref, beta_ref, o0_ref, o1_ref):
    h = jnp.concatenate([h0_ref[...], h1_ref[...]], axis=1)
    p = jnp.concatenate([p0_ref[:N, :], p1_ref[:N, :]], axis=1)
    comb = (jnp.dot(h, vw_ref[...], preferred_element_type=jnp.float32)
            + jnp.dot(p, aw_ref[...], preferred_element_type=jnp.float32)
            + bias_ref[...])
    h_new = _bn_relu(comb, gamma_ref[...], beta_ref[...])
    o0_ref[...] = h_new[:, :HALF]
    o1_ref[...] = h_new[:, HALF:]


_combine = pl.pallas_call(
    _combine_body,
    out_shape=(
        jax.ShapeDtypeStruct((N, HALF), jnp.float32),
        jax.ShapeDtypeStruct((N, HALF), jnp.float32),
    ),
)


def _combine_head_body(h0_ref, h1_ref, p0_ref, p1_ref, vw_ref, aw_ref,
                       bias_ref, gamma_ref, beta_ref, pw_ref, pb_ref, o_ref):
    h = jnp.concatenate([h0_ref[...], h1_ref[...]], axis=1)
    p = jnp.concatenate([p0_ref[:N, :], p1_ref[:N, :]], axis=1)
    comb = (jnp.dot(h, vw_ref[...], preferred_element_type=jnp.float32)
            + jnp.dot(p, aw_ref[...], preferred_element_type=jnp.float32)
            + bias_ref[...])
    h_new = _bn_relu(comb, gamma_ref[...], beta_ref[...])
    o_ref[...] = (jnp.dot(h_new, pw_ref[...],
                          preferred_element_type=jnp.float32) + pb_ref[...])


_combine_head = pl.pallas_call(
    _combine_head_body,
    out_shape=jax.ShapeDtypeStruct((N, OUT), jnp.float32),
)


def kernel(x, edge_index, V_w, V_b, A_w, A_b, bn_gamma, bn_beta, pred_w, pred_b):
    dst2d = edge_index[0].reshape(NCHUNK, K)
    src2d = edge_index[1].reshape(NCHUNK, K)

    pa, pb = _segsum0(x, src2d, dst2d)
    bias0 = (V_b[0] + A_b[0]).reshape(1, H)
    h0, h1 = _combine0(x, pa, pb, V_w[0, :HALF, :], A_w[0, :HALF, :], bias0,
                       bn_gamma[0].reshape(1, H), bn_beta[0].reshape(1, H))
    p0, p1 = _segsum(h0, h1, src2d, dst2d)
    bias = (V_b[1] + A_b[1]).reshape(1, H)
    h0, h1 = _combine(h0, h1, p0, p1, V_w[1], A_w[1], bias,
                      bn_gamma[1].reshape(1, H), bn_beta[1].reshape(1, H))
    p0, p1 = _segsum(h0, h1, src2d, dst2d)
    bias = (V_b[2] + A_b[2]).reshape(1, H)
    return _combine_head(h0, h1, p0, p1, V_w[2], A_w[2], bias,
                         bn_gamma[2].reshape(1, H), bn_beta[2].reshape(1, H),
                         pred_w, pred_b.reshape(1, OUT))
